# Initial kernel scaffold; baseline (speedup 1.0000x reference)
import jax
import jax.numpy as jnp
from jax.experimental import pallas as pl


def kernel(enc_output, W_ctc, b_ctc):
    raise NotImplementedError("write your pallas kernel here")


# speedup vs baseline: 17.3354x; 17.3354x over previous
import functools

import jax
import jax.numpy as jnp
from jax.experimental import pallas as pl
from jax.experimental.pallas import tpu as pltpu

T = 8192
D = 128
V = 10000
VP = 10240
PRE_BEAM = 30
BLANK = 0
R = 256
NEG_PAD = -3.0e38


def _to_key(x):
    i = jax.lax.bitcast_convert_type(x, jnp.int32)
    return jnp.where(i < 0, i ^ jnp.int32(0x7FFFFFFF), i)


def _from_key(k):
    i = jnp.where(k < 0, k ^ jnp.int32(0x7FFFFFFF), k)
    return jax.lax.bitcast_convert_type(i, jnp.float32)


def _body(enc_ref, w_ref, b_ref, out_ref, keys_ref):
    logits = (
        jnp.dot(enc_ref[:], w_ref[:], preferred_element_type=jnp.float32)
        + b_ref[:]
    )
    m = jnp.max(logits, axis=1, keepdims=True)
    sh = logits - m
    se = jnp.sum(jnp.exp(sh), axis=1, keepdims=True)
    lpz = sh - jnp.log(se)
    keys_ref[:] = _to_key(lpz)

    lo0 = jnp.full((R, 1), jnp.iinfo(jnp.int32).min, jnp.int32)
    hi0 = jnp.full((R, 1), jnp.iinfo(jnp.int32).max, jnp.int32)

    def it(_, c):
        lo, hi = c
        mid = (lo >> 1) + (hi >> 1) + (lo & hi & 1)
        cnt = jnp.sum(
            (keys_ref[:] >= mid).astype(jnp.int32), axis=1, keepdims=True
        )
        ge = cnt >= PRE_BEAM
        return jnp.where(ge, mid, lo), jnp.where(ge, hi, mid)

    lo, _ = jax.lax.fori_loop(0, 32, it, (lo0, hi0))

    keys = keys_ref[:]
    col = jax.lax.broadcasted_iota(jnp.int32, (R, VP), 1)
    mask = (keys >= lo) | (col == BLANK)
    out = jnp.where(mask, _from_key(keys), -jnp.inf)
    out_ref[:] = out[:, :V]


@jax.jit
def kernel(enc_output, W_ctc, b_ctc):
    w_pad = jnp.concatenate(
        [W_ctc, jnp.zeros((D, VP - V), jnp.float32)], axis=1
    )
    b_pad = jnp.concatenate(
        [b_ctc, jnp.full((VP - V,), NEG_PAD, jnp.float32)]
    ).reshape(1, VP)
    grid = (T // R,)
    return pl.pallas_call(
        _body,
        grid=grid,
        in_specs=[
            pl.BlockSpec((R, D), lambda i: (i, 0)),
            pl.BlockSpec((D, VP), lambda i: (0, 0)),
            pl.BlockSpec((1, VP), lambda i: (0, 0)),
        ],
        out_specs=pl.BlockSpec((R, V), lambda i: (i, 0)),
        out_shape=jax.ShapeDtypeStruct((T, V), jnp.float32),
        scratch_shapes=[pltpu.VMEM((R, VP), jnp.int32)],
    )(enc_output, w_pad, b_pad)
